# Initial kernel scaffold; baseline (speedup 1.0000x reference)
import jax
import jax.numpy as jnp
from jax.experimental import pallas as pl


def kernel(feature, w_logits, b_logits, w_reg, b_reg):
    raise NotImplementedError("write your pallas kernel here")


# speedup vs baseline: 9.4722x; 9.4722x over previous
import functools

import jax
import jax.numpy as jnp
from jax import lax
from jax.experimental import pallas as pl
from jax.experimental.pallas import tpu as pltpu
from jax.experimental.pallas import tpu_sc as plsc

N = 16384
C = 192
TOPK = 5000
NPAD = 5120
BLK = 128
NBLK = NPAD // BLK
MAXOUT = 512
MIN_SCORE = 0.2
THR = 0.015625
SCALE = 4.0

_MBLK = 2048


def _head_body(f_ref, w_ref, b_ref, out_ref):
    acc = jnp.dot(f_ref[...], w_ref[...],
                  preferred_element_type=jnp.float32) + b_ref[...]
    i = pl.program_id(0) * _MBLK + lax.broadcasted_iota(
        jnp.int32, (_MBLK, 1), 0)
    r = (i // 128).astype(jnp.float32)
    c = (i % 128).astype(jnp.float32)
    loc0 = (r + 0.5) * SCALE + acc[:, 1:2] * SCALE
    loc1 = (c + 0.5) * SCALE + acc[:, 2:3] * SCALE
    out_ref[:, 0:1] = acc[:, 0:1]
    out_ref[:, 1:2] = loc0
    out_ref[:, 2:3] = loc1
    out_ref[:, 3:8] = jnp.zeros((_MBLK, 5), jnp.float32)


def _head(flat, w8, b8):
    return pl.pallas_call(
        _head_body,
        grid=(N // _MBLK,),
        in_specs=[
            pl.BlockSpec((_MBLK, C), lambda k: (k, 0)),
            pl.BlockSpec((C, 8), lambda k: (0, 0)),
            pl.BlockSpec((1, 8), lambda k: (0, 0)),
        ],
        out_specs=pl.BlockSpec((_MBLK, 8), lambda k: (k, 0)),
        out_shape=jax.ShapeDtypeStruct((N, 8), jnp.float32),
    )(flat, w8, b8)


_IBLK = 64


def _rank_body(scol_ref, srow_ref, out_ref):
    si = scol_ref[...]
    sa = srow_ref[...]
    gt = (sa > si).astype(jnp.int32)
    jidx = lax.broadcasted_iota(jnp.int32, (_IBLK, N), 1)
    iidx = (pl.program_id(0) * _IBLK
            + lax.broadcasted_iota(jnp.int32, (_IBLK, N), 0))
    tie = jnp.where((sa == si) & (jidx < iidx), 1, 0)
    out_ref[...] = jnp.sum(gt + tie, axis=1, keepdims=True)


def _ranks(scores):
    return pl.pallas_call(
        _rank_body,
        grid=(N // _IBLK,),
        in_specs=[
            pl.BlockSpec((_IBLK, 1), lambda k: (k, 0)),
            pl.BlockSpec((1, N), lambda k: (0, 0)),
        ],
        out_specs=pl.BlockSpec((_IBLK, 1), lambda k: (k, 0)),
        out_shape=jax.ShapeDtypeStruct((N, 1), jnp.int32),
    )(scores.reshape(N, 1), scores.reshape(1, N))


_NW = 32
_CH = N // _NW
_NJ = _CH // 128


def _scatter_body(rank_hbm, s_hbm, c0_hbm, c1_hbm,
                  out_s, out_c0, out_c1,
                  idx_v, sv, c0v, c1v, sem):
    wid = lax.axis_index("s") * 2 + lax.axis_index("c")
    pltpu.sync_copy(rank_hbm.at[wid], idx_v)
    pltpu.sync_copy(s_hbm.at[wid], sv)
    pltpu.sync_copy(c0_hbm.at[wid], c0v)
    pltpu.sync_copy(c1_hbm.at[wid], c1v)
    copies = []
    for j in range(_NJ):
        copies.append(pltpu.async_copy(sv.at[j], out_s.at[idx_v.at[j]], sem))
        copies.append(pltpu.async_copy(c0v.at[j], out_c0.at[idx_v.at[j]], sem))
        copies.append(pltpu.async_copy(c1v.at[j], out_c1.at[idx_v.at[j]], sem))
    for cp in copies:
        cp.wait()


def _scatter_by_rank(ranks, s, c0, c1):
    mesh = plsc.VectorSubcoreMesh(core_axis_name="c", subcore_axis_name="s")
    f = functools.partial(
        pl.kernel,
        mesh=mesh,
        out_type=[jax.ShapeDtypeStruct((N,), jnp.float32)] * 3,
        scratch_types=[pltpu.VMEM((_NJ, 128), jnp.int32)]
        + [pltpu.VMEM((_NJ, 128), jnp.float32)] * 3
        + [pltpu.SemaphoreType.DMA],
    )(_scatter_body)
    r3 = ranks.reshape(_NW, _NJ, 128)
    return f(r3, s.reshape(_NW, _NJ, 128), c0.reshape(_NW, _NJ, 128),
             c1.reshape(_NW, _NJ, 128))


def _nms_body(s_r, xr, yr, xc, yc, xa, ya, keep_out, supp_ref):
    k = pl.program_id(0)

    @pl.when(k == 0)
    def _():
        supp_ref[...] = jnp.zeros((NBLK, BLK), jnp.float32)

    rowsel = lax.broadcasted_iota(jnp.int32, (NBLK, BLK), 0) == k
    supp_row = jnp.sum(jnp.where(rowsel, supp_ref[...], 0.0), axis=0,
                       keepdims=True)

    s = s_r[0]
    pos_row = (k * BLK
               + lax.broadcasted_iota(jnp.int32, (1, BLK), 1))
    keep0 = ((s > MIN_SCORE) & (pos_row < TOPK)
             & (supp_row <= 0.0)).astype(jnp.float32)

    x_r = xr[0]
    y_r = yr[0]
    x_c = xc[...]
    y_c = yc[...]
    sq_r = x_r * x_r + y_r * y_r
    sq_c = x_c * x_c + y_c * y_c

    lhs = jnp.concatenate([x_c, y_c], axis=1)
    rhs = jnp.concatenate([x_r, y_r], axis=0)
    dot = jnp.dot(lhs, rhs, preferred_element_type=jnp.float32)
    d2 = (sq_c + sq_r) - 2.0 * dot
    jgt = (lax.broadcasted_iota(jnp.int32, (BLK, BLK), 1)
           > lax.broadcasted_iota(jnp.int32, (BLK, BLK), 0))
    skill = ((jnp.maximum(d2, 0.0) * THR < 1.0) & jgt).astype(jnp.float32)
    rowio = lax.broadcasted_iota(jnp.int32, (BLK, BLK), 0)
    oneh = lax.broadcasted_iota(jnp.int32, (1, BLK), 1)

    def body(i, kp):
        ki = jnp.sum(jnp.where(oneh == i, kp, 0.0)) > 0.0
        rowi = jnp.sum(jnp.where(rowio == i, skill, 0.0), axis=0,
                       keepdims=True)
        return jnp.where(ki & (rowi > 0.0), 0.0, kp)

    keep = lax.fori_loop(0, BLK, body, keep0)
    keep_out[0] = keep

    kc = keep.reshape(BLK, 1)
    for m in range(NBLK):
        x_m = xa[m:m + 1, :]
        y_m = ya[m:m + 1, :]
        sq_m = x_m * x_m + y_m * y_m
        rhs_m = jnp.concatenate([x_m, y_m], axis=0)
        dot_m = jnp.dot(lhs, rhs_m, preferred_element_type=jnp.float32)
        d2_m = (sq_c + sq_m) - 2.0 * dot_m
        sup_m = jnp.where(jnp.maximum(d2_m, 0.0) * THR < 1.0, kc, 0.0)
        any_m = jnp.sum(sup_m, axis=0, keepdims=True)
        gate = jnp.where(jnp.int32(m) > k, 1.0, 0.0)
        supp_ref[m:m + 1, :] = jnp.maximum(supp_ref[m:m + 1, :], any_m * gate)


def _nms(ss, xx, yy):
    row3 = lambda a: a.reshape(NBLK, 1, BLK)
    row = lambda a: a.reshape(NBLK, BLK)
    col = lambda a: a.reshape(NPAD, 1)
    out = pl.pallas_call(
        _nms_body,
        grid=(NBLK,),
        in_specs=[
            pl.BlockSpec((1, 1, BLK), lambda k: (k, 0, 0)),
            pl.BlockSpec((1, 1, BLK), lambda k: (k, 0, 0)),
            pl.BlockSpec((1, 1, BLK), lambda k: (k, 0, 0)),
            pl.BlockSpec((BLK, 1), lambda k: (k, 0)),
            pl.BlockSpec((BLK, 1), lambda k: (k, 0)),
            pl.BlockSpec((NBLK, BLK), lambda k: (0, 0)),
            pl.BlockSpec((NBLK, BLK), lambda k: (0, 0)),
        ],
        out_specs=pl.BlockSpec((1, 1, BLK), lambda k: (k, 0, 0)),
        out_shape=jax.ShapeDtypeStruct((NBLK, 1, BLK), jnp.float32),
        scratch_shapes=[pltpu.VMEM((NBLK, BLK), jnp.float32)],
    )(row3(ss), row3(xx), row3(yy), col(xx), col(yy), row(xx), row(yy))
    return out.reshape(NBLK, BLK)


def _select_body(keep_ref, sc_ref, xc_ref, yc_ref, os_ref, ox_ref, oy_ref):
    kp = keep_ref[...]
    lt = (lax.broadcasted_iota(jnp.int32, (BLK, BLK), 0)
          < lax.broadcasted_iota(jnp.int32, (BLK, BLK), 1)).astype(jnp.float32)
    rowcum = jnp.dot(kp, lt, preferred_element_type=jnp.float32)
    rowtot = jnp.sum(kp, axis=1, keepdims=True)
    tri = (lax.broadcasted_iota(jnp.int32, (NBLK, NBLK), 1)
           < lax.broadcasted_iota(jnp.int32, (NBLK, NBLK), 0)).astype(
               jnp.float32)
    rowpref = jnp.dot(tri, rowtot, preferred_element_type=jnp.float32)
    pos = rowpref + rowcum
    sel = (kp > 0.0) & (pos < float(MAXOUT))
    cnt = jnp.sum(kp)
    p_col = lax.broadcasted_iota(jnp.int32, (MAXOUT, 1), 0).astype(jnp.float32)
    acc_s = jnp.zeros((MAXOUT, 1), jnp.float32)
    acc_x = jnp.zeros((MAXOUT, 1), jnp.float32)
    acc_y = jnp.zeros((MAXOUT, 1), jnp.float32)
    for r in range(NBLK):
        oh = jnp.where((pos[r:r + 1, :] == p_col) & sel[r:r + 1, :],
                       1.0, 0.0)
        acc_s = acc_s + jnp.dot(oh, sc_ref[r * BLK:(r + 1) * BLK, :],
                                preferred_element_type=jnp.float32)
        acc_x = acc_x + jnp.dot(oh, xc_ref[r * BLK:(r + 1) * BLK, :],
                                preferred_element_type=jnp.float32)
        acc_y = acc_y + jnp.dot(oh, yc_ref[r * BLK:(r + 1) * BLK, :],
                                preferred_element_type=jnp.float32)
    filled = p_col < cnt
    os_ref[...] = jnp.where(filled, acc_s, -1.0)
    ox_ref[...] = jnp.where(filled, acc_x, -1.0)
    oy_ref[...] = jnp.where(filled, acc_y, -1.0)


def _compact(keep, ss, xx, yy):
    col = lambda a: a.reshape(NPAD, 1)
    outs = pl.pallas_call(
        _select_body,
        out_shape=[jax.ShapeDtypeStruct((MAXOUT, 1), jnp.float32)] * 3,
    )(keep, col(ss), col(xx), col(yy))
    return tuple(o.reshape(MAXOUT) for o in outs)


def kernel(feature, w_logits, b_logits, w_reg, b_reg):
    flat = feature.reshape(N, C)
    w8 = jnp.zeros((C, 8), jnp.float32)
    w8 = w8.at[:, 0:1].set(w_logits).at[:, 1:3].set(w_reg)
    b8 = jnp.zeros((1, 8), jnp.float32)
    b8 = b8.at[0, 0].set(b_logits[0]).at[0, 1:3].set(b_reg)

    head = _head(flat, w8, b8)
    logits = head[:, 0]
    c0 = head[:, 1]
    c1 = head[:, 2]
    scores = jax.nn.sigmoid(logits)

    ranks = _ranks(scores).reshape(N)
    ss, sc0, sc1 = _scatter_by_rank(ranks, scores, c0, c1)

    keep = _nms(ss[:NPAD], sc0[:NPAD], sc1[:NPAD])
    out_s, out_c0, out_c1 = _compact(keep, ss[:NPAD], sc0[:NPAD], sc1[:NPAD])
    return (out_s, jnp.stack([out_c0, out_c1], axis=-1))
